# Initial kernel scaffold; baseline (speedup 1.0000x reference)
import jax
import jax.numpy as jnp
from jax.experimental import pallas as pl


def kernel(logits_student, logits_teacher, target):
    raise NotImplementedError("write your pallas kernel here")


# speedup vs baseline: 25.2421x; 25.2421x over previous
import functools

import jax
import jax.numpy as jnp
from jax.experimental import pallas as pl
from jax.experimental.pallas import tpu as pltpu

_T = 4.0
_ALPHA = 1.0
_BETA = 8.0
_TOPK = 100
_C = 1000
_R = 256


def _dkd_block(s_ref, t_ref, g_ref, out_ref):
    i = pl.program_id(0)
    sraw = s_ref[...]
    traw = t_ref[...]
    g = g_ref[0, 0, :]
    rows = sraw.shape[0]

    s = sraw * jnp.float32(1.0 / _T)
    t = traw * jnp.float32(1.0 / _T)

    tz = jnp.where(traw == 0.0, jnp.float32(0.0), traw)
    bits = jax.lax.bitcast_convert_type(tz, jnp.int32)
    key = bits ^ (jax.lax.shift_right_arithmetic(bits, 31) & jnp.int32(0x7FFFFFFF))

    p = jnp.full((rows, 1), jnp.int32(-2147483648), jnp.int32)
    cand0 = jnp.zeros_like(p)
    cnt0 = jnp.sum((key >= cand0).astype(jnp.int32), axis=1, keepdims=True)
    p = jnp.where(cnt0 >= _TOPK, cand0, p)
    for b in range(30, -1, -1):
        cand = p + jnp.int32(1 << b)
        cnt = jnp.sum((key >= cand).astype(jnp.int32), axis=1, keepdims=True)
        p = jnp.where(cnt >= _TOPK, cand, p)

    n_hi = jnp.sum((key > p).astype(jnp.int32), axis=1, keepdims=True)
    r_need = jnp.int32(_TOPK) - n_hi
    eq = key == p
    col = jax.lax.broadcasted_iota(jnp.int32, (rows, _C), 1)
    lo = jnp.zeros((rows, 1), jnp.int32)
    hi = jnp.full((rows, 1), jnp.int32(_C - 1), jnp.int32)
    for _ in range(10):
        mid = (lo + hi) >> 1
        cnt = jnp.sum((eq & (col <= mid)).astype(jnp.int32), axis=1, keepdims=True)
        take = cnt >= r_need
        hi = jnp.where(take, mid, hi)
        lo = jnp.where(take, lo, mid + 1)
    topk_mask = (key > p) | (eq & (col <= lo))

    gtm = col == g[:, None]
    other = topk_mask & jnp.logical_not(gtm)

    ms = jnp.max(s, axis=1, keepdims=True)
    mt = jnp.max(t, axis=1, keepdims=True)
    es = jnp.exp(s - ms)
    et = jnp.exp(t - mt)
    zs = jnp.sum(es, axis=1, keepdims=True)
    zt = jnp.sum(et, axis=1, keepdims=True)
    lse_s = ms + jnp.log(zs)
    lse_t = mt + jnp.log(zt)
    s_g = jnp.sum(jnp.where(gtm, s, 0.0), axis=1, keepdims=True)
    t_g = jnp.sum(jnp.where(gtm, t, 0.0), axis=1, keepdims=True)
    lps1 = s_g - lse_s
    lpt1 = t_g - lse_t
    ps2 = jnp.sum(jnp.where(other, es, 0.0), axis=1, keepdims=True) / zs
    pt2 = jnp.sum(jnp.where(other, et, 0.0), axis=1, keepdims=True) / zt
    tckd = (jnp.exp(lpt1) * (lpt1 - lps1)
            + pt2 * (jnp.log(pt2) - jnp.log(ps2)))

    neg = jnp.float32(-1e30)
    mzt = jnp.max(jnp.where(other, t, neg), axis=1, keepdims=True)
    mzs = jnp.max(jnp.where(other, s, neg), axis=1, keepdims=True)
    eot = jnp.where(other, jnp.exp(t - mzt), 0.0)
    eos = jnp.where(other, jnp.exp(s - mzs), 0.0)
    zot = jnp.sum(eot, axis=1, keepdims=True)
    zos = jnp.sum(eos, axis=1, keepdims=True)
    qt = eot / zot
    nckd = (jnp.sum(qt * (t - s), axis=1, keepdims=True)
            + (mzs + jnp.log(zos)) - (mzt + jnp.log(zot)))

    block_loss = jnp.sum(_ALPHA * tckd + _BETA * nckd, axis=0, keepdims=True)

    @pl.when(i == 0)
    def _init():
        out_ref[...] = jnp.zeros((1, 1), jnp.float32)

    out_ref[...] += block_loss


@jax.jit
def kernel(logits_student, logits_teacher, target):
    bsz, c = logits_teacher.shape
    nblk = bsz // _R
    tgt = target.reshape(nblk, 1, _R)
    out = pl.pallas_call(
        _dkd_block,
        grid=(nblk,),
        in_specs=[
            pl.BlockSpec((_R, c), lambda i: (i, 0)),
            pl.BlockSpec((_R, c), lambda i: (i, 0)),
            pl.BlockSpec((1, 1, _R), lambda i: (i, 0, 0)),
        ],
        out_specs=pl.BlockSpec((1, 1), lambda i: (0, 0)),
        out_shape=jax.ShapeDtypeStruct((1, 1), jnp.float32),
        compiler_params=pltpu.CompilerParams(
            dimension_semantics=("arbitrary",),
        ),
    )(logits_student, logits_teacher, tgt)
    return out[0, 0] * jnp.float32(_T * _T / bsz)
